# Initial kernel scaffold; baseline (speedup 1.0000x reference)
import jax
import jax.numpy as jnp
from jax.experimental import pallas as pl


def kernel(x, edge_attr, agent_state, edge_index, pool_batch, W_gat, att_src, att_dst, W_edge, att_edge, b_gat, W1, b1, W2, b2, Wv1, bv1, Wv2, bv2, Wa1, ba1, Wa2, ba2):
    raise NotImplementedError("write your pallas kernel here")


# speedup vs baseline: 1.3483x; 1.3483x over previous
import functools

import jax
import jax.numpy as jnp
from jax.experimental import pallas as pl
from jax.experimental.pallas import tpu as pltpu

N = 100000
NPAD = 100352
B = 128
NB = 2048
GRID = NPAD // NB


def _tail_pool_body(hgat_ref, pb_ref, W1_ref, b1_ref, pooled_ref, counts_ref):
    i = pl.program_id(0)

    @pl.when(i == 0)
    def _init():
        pooled_ref[...] = jnp.zeros_like(pooled_ref)
        counts_ref[...] = jnp.zeros_like(counts_ref)

    h1 = jnp.maximum(hgat_ref[...] @ W1_ref[...] + b1_ref[...], 0.0)
    pb = pb_ref[0, pl.ds(i * NB, NB)]
    onehot = (pb[None, :] == jax.lax.broadcasted_iota(jnp.int32, (B, NB), 0)
              ).astype(jnp.float32)
    pooled_ref[...] += onehot @ h1
    counts_ref[...] += onehot @ jnp.ones((NB, 1), jnp.float32)


def _head_body(pooled_ref, counts_ref, ag_ref, W2_ref, b2_ref,
               Wv1_ref, bv1_ref, Wv2_ref, bv2_ref,
               Wa1_ref, ba1_ref, Wa2_ref, ba2_ref, out_ref):
    counts = jnp.maximum(counts_ref[...], 1.0)
    pooled = pooled_ref[...] / counts
    ag = jnp.maximum(ag_ref[...] @ W2_ref[...] + b2_ref[...], 0.0)
    z = jnp.concatenate([pooled, ag], axis=-1)
    v = jnp.maximum(z @ Wv1_ref[...] + bv1_ref[...], 0.0)
    value = v @ Wv2_ref[...] + bv2_ref[...]
    a = jnp.maximum(z @ Wa1_ref[...] + ba1_ref[...], 0.0)
    adv = a @ Wa2_ref[...] + ba2_ref[...]
    out_ref[...] = value + adv - jnp.mean(adv)


def _dense_tail(hgat, pool_batch, agent_state, W1, b1, W2, b2,
                Wv1, bv1, Wv2, bv2, Wa1, ba1, Wa2, ba2):
    hgat_pad = jnp.pad(hgat, ((0, NPAD - N), (0, 0)))
    pb_pad = jnp.pad(pool_batch, (0, NPAD - N), constant_values=-1)
    pooled_sum, counts = pl.pallas_call(
        _tail_pool_body,
        grid=(GRID,),
        in_specs=[
            pl.BlockSpec((NB, 64), lambda i: (i, 0)),
            pl.BlockSpec((1, NPAD), lambda i: (0, 0)),
            pl.BlockSpec((64, 128), lambda i: (0, 0)),
            pl.BlockSpec((1, 128), lambda i: (0, 0)),
        ],
        out_specs=[
            pl.BlockSpec((B, 128), lambda i: (0, 0)),
            pl.BlockSpec((B, 1), lambda i: (0, 0)),
        ],
        out_shape=[
            jax.ShapeDtypeStruct((B, 128), jnp.float32),
            jax.ShapeDtypeStruct((B, 1), jnp.float32),
        ],
    )(hgat_pad, pb_pad.reshape(1, NPAD), W1, b1.reshape(1, 128))

    out = pl.pallas_call(
        _head_body,
        out_shape=jax.ShapeDtypeStruct((B, 8), jnp.float32),
    )(pooled_sum, counts, agent_state, W2, b2.reshape(1, 64),
      Wv1, bv1.reshape(1, 128), Wv2, bv2.reshape(1, 1),
      Wa1, ba1.reshape(1, 128), Wa2, ba2.reshape(1, 8))
    return out


def kernel(x, edge_attr, agent_state, edge_index, pool_batch, W_gat, att_src,
           att_dst, W_edge, att_edge, b_gat, W1, b1, W2, b2, Wv1, bv1, Wv2,
           bv2, Wa1, ba1, Wa2, ba2):
    n = x.shape[0]
    src = edge_index[0]
    dst = edge_index[1]
    c_edge = jnp.dot(W_edge[0], att_edge)
    h = x @ W_gat
    a_src = h @ att_src
    a_dst = h @ att_dst
    a_edge = edge_attr[:, 0] * c_edge

    alpha = a_src[src] + a_dst[dst] + a_edge
    alpha = jnp.where(alpha >= 0, alpha, 0.2 * alpha)
    w = jnp.exp(alpha)
    denom = jax.ops.segment_sum(w, dst, num_segments=n)
    acc = jax.ops.segment_sum(w[:, None] * h[src], dst, num_segments=n)

    a_self = a_src + a_dst + jnp.mean(edge_attr) * c_edge
    a_self = jnp.where(a_self >= 0, a_self, 0.2 * a_self)
    w_self = jnp.exp(a_self)
    denom = denom + w_self
    acc = acc + w_self[:, None] * h

    hgat = acc / denom[:, None] + b_gat[None, :]

    return _dense_tail(hgat, pool_batch, agent_state, W1, b1, W2, b2,
                       Wv1, bv1, Wv2, bv2, Wa1, ba1, Wa2, ba2)


# speedup vs baseline: 4.8581x; 3.6032x over previous
import functools

import jax
import jax.numpy as jnp
from jax import lax
from jax.experimental import pallas as pl
from jax.experimental.pallas import tpu as pltpu
from jax.experimental.pallas import tpu_sc as plsc

N = 100000
NPAD = 100352
B = 128
NB = 2048
GRID = NPAD // NB

E = 3200000
NW = 32
EP = 3276800
EW = EP // NW
BE = 2048
NBLK = EW // BE

_sc_mesh = plsc.VectorSubcoreMesh(core_axis_name="c", subcore_axis_name="s")
_sc_params = pltpu.CompilerParams(needs_layout_passes=False,
                                  use_tc_tiling_on_sc=False)


def _wid():
    return lax.axis_index("s") * 2 + lax.axis_index("c")


@functools.partial(
    pl.kernel, mesh=_sc_mesh, compiler_params=_sc_params,
    out_type=[jax.ShapeDtypeStruct((EP,), jnp.float32),
              jax.ShapeDtypeStruct((NW, 16), jnp.float32)],
    scratch_types=[
        pltpu.VMEM((NPAD,), jnp.float32),
        pltpu.VMEM((BE,), jnp.int32),
        pltpu.VMEM((BE,), jnp.float32),
        pltpu.VMEM((BE,), jnp.float32),
        pltpu.VMEM((16,), jnp.float32),
    ],
)
def _k1(src_hbm, ea_hbm, asrc_hbm, t_hbm, easum_hbm,
        table_v, src_v, ea_v, t_v, sum_v):
    wid = _wid()
    base = wid * EW
    pltpu.sync_copy(asrc_hbm, table_v)

    def blk(b, acc):
        off = base + b * BE
        pltpu.sync_copy(src_hbm.at[pl.ds(off, BE)], src_v)
        pltpu.sync_copy(ea_hbm.at[pl.ds(off, BE)], ea_v)

        def grp(g, a):
            idx = src_v[pl.ds(g * 16, 16)]
            vals = plsc.load_gather(table_v, [idx])
            ea16 = ea_v[pl.ds(g * 16, 16)]
            t_v[pl.ds(g * 16, 16)] = vals + ea16
            return a + ea16

        acc = lax.fori_loop(0, BE // 16, grp, acc)
        pltpu.sync_copy(t_v, t_hbm.at[pl.ds(off, BE)])
        return acc

    acc = lax.fori_loop(0, NBLK, blk, jnp.zeros((16,), jnp.float32))
    sum_v[...] = acc
    pltpu.sync_copy(sum_v, easum_hbm.at[wid])


@functools.partial(
    pl.kernel, mesh=_sc_mesh, compiler_params=_sc_params,
    out_type=jax.ShapeDtypeStruct((EP,), jnp.float32),
    scratch_types=[
        pltpu.VMEM((NPAD,), jnp.float32),
        pltpu.VMEM((BE,), jnp.int32),
        pltpu.VMEM((BE,), jnp.float32),
        pltpu.VMEM((BE,), jnp.float32),
    ],
)
def _k2(dst_hbm, t_hbm, adst_hbm, w_hbm, table_v, dst_v, t_v, w_v):
    base = _wid() * EW
    pltpu.sync_copy(adst_hbm, table_v)

    def blk(b, _):
        off = base + b * BE
        pltpu.sync_copy(dst_hbm.at[pl.ds(off, BE)], dst_v)
        pltpu.sync_copy(t_hbm.at[pl.ds(off, BE)], t_v)

        def grp(g, _):
            idx = dst_v[pl.ds(g * 16, 16)]
            alpha = plsc.load_gather(table_v, [idx]) + t_v[pl.ds(g * 16, 16)]
            alpha = jnp.where(alpha >= 0, alpha, 0.2 * alpha)
            w_v[pl.ds(g * 16, 16)] = jnp.exp(alpha)
            return 0

        lax.fori_loop(0, BE // 16, grp, 0)
        pltpu.sync_copy(w_v, w_hbm.at[pl.ds(off, BE)])
        return 0

    lax.fori_loop(0, NBLK, blk, 0)


CH = 26624
TROWS = CH // 16
NCHUNK = 4
KQ = 16


@functools.partial(
    pl.kernel, mesh=_sc_mesh, compiler_params=_sc_params,
    out_type=jax.ShapeDtypeStruct((NCHUNK * CH, 64), jnp.float32),
    scratch_types=[
        pltpu.VMEM((BE,), jnp.int32),
        pltpu.VMEM((BE,), jnp.int32),
        pltpu.VMEM((BE,), jnp.float32),
        pltpu.VMEM((KQ, 64), jnp.float32),
        pltpu.VMEM((KQ, 64), jnp.float32),
        pltpu.VMEM((KQ,), jnp.float32),
        pltpu.VMEM((KQ,), jnp.int32),
        pltpu.VMEM_SHARED((CH, 64), jnp.float32),
        pltpu.SemaphoreType.DMA,
    ],
)
def _k3(src_hbm, dst_hbm, w_hbm, h_hbm, acc_hbm,
        src_v, dst_v, w_v, rows_v, zero_v, cw_v, dl_v, acc_sh, sem):
    c = lax.axis_index("c")
    s = lax.axis_index("s")
    base = s * (EP // 16)

    def zrow(i, _):
        for cc in range(4):
            zero_v[i, pl.ds(cc * 16, 16)] = jnp.zeros((16,), jnp.float32)
        return 0

    lax.fori_loop(0, KQ, zrow, 0)

    def scan(lo):
        def blk(b, _):
            off = base + b * BE
            pltpu.sync_copy(src_hbm.at[pl.ds(off, BE)], src_v)
            pltpu.sync_copy(dst_hbm.at[pl.ds(off, BE)], dst_v)
            pltpu.sync_copy(w_hbm.at[pl.ds(off, BE)], w_v)

            def chunk(q, _):
                qo = q * KQ

                def grp(g, _):
                    i = qo + g * 16
                    d16 = dst_v[pl.ds(i, 16)]
                    w16 = w_v[pl.ds(i, 16)]
                    dloc = d16 - lo
                    m = (dloc >= 0) & (dloc < CH)
                    cw_v[pl.ds(g * 16, 16)] = jnp.where(m, w16, 0.0)
                    dl_v[pl.ds(g * 16, 16)] = jnp.where(m, dloc, 0)
                    return 0

                lax.fori_loop(0, KQ // 16, grp, 0)
                pltpu.async_copy(
                    h_hbm.at[src_v.at[pl.ds(qo, KQ)]], rows_v, sem).wait()

                def scale(k, _):
                    wb = plsc.load_gather(
                        cw_v, [jnp.zeros((16,), jnp.int32) + k])
                    for cc in range(4):
                        seg = rows_v[k, pl.ds(cc * 16, 16)]
                        rows_v[k, pl.ds(cc * 16, 16)] = seg * wb
                    return 0

                lax.fori_loop(0, KQ, scale, 0)
                pltpu.sync_copy(rows_v, acc_sh.at[dl_v], add=True)
                return 0

            lax.fori_loop(0, BE // KQ, chunk, 0)
            return 0

        lax.fori_loop(0, (EP // 16) // BE, blk, 0)

    def writeback(lo):
        def wb_loop(z, _):
            row0 = s * TROWS + z * KQ
            pltpu.sync_copy(acc_sh.at[pl.ds(row0, KQ)],
                            acc_hbm.at[pl.ds(lo + row0, KQ)])
            return 0

        lax.fori_loop(0, TROWS // KQ, wb_loop, 0)

    for p in range(2):
        def zacc(z, _):
            pltpu.sync_copy(zero_v, acc_sh.at[pl.ds(s * TROWS + z * KQ, KQ)])
            return 0

        lax.fori_loop(0, TROWS // KQ, zacc, 0)
        plsc.subcore_barrier()

        for cval in range(2):
            @pl.when(c == cval)
            def _(p=p, cval=cval):
                scan((2 * p + cval) * CH)

        plsc.subcore_barrier()

        for cval in range(2):
            @pl.when(c == cval)
            def _(p=p, cval=cval):
                writeback((2 * p + cval) * CH)

        plsc.subcore_barrier()


@functools.partial(
    pl.kernel, mesh=_sc_mesh, compiler_params=_sc_params,
    out_type=jax.ShapeDtypeStruct((NW, NPAD), jnp.float32),
    scratch_types=[
        pltpu.VMEM((NPAD,), jnp.float32),
        pltpu.VMEM((BE,), jnp.int32),
        pltpu.VMEM((BE,), jnp.float32),
    ],
)
def _k2d(dst_hbm, w_hbm, dpart_hbm, den_v, dst_v, w_v):
    wid = _wid()
    base = wid * EW

    def zero(i, _):
        den_v[pl.ds(i * 16, 16)] = jnp.zeros((16,), jnp.float32)
        return 0

    lax.fori_loop(0, NPAD // 16, zero, 0)

    def blk(b, _):
        off = base + b * BE
        pltpu.sync_copy(dst_hbm.at[pl.ds(off, BE)], dst_v)
        pltpu.sync_copy(w_hbm.at[pl.ds(off, BE)], w_v)

        def grp(g, _):
            idx = dst_v[pl.ds(g * 16, 16)]
            plsc.addupdate_scatter(den_v, [idx], w_v[pl.ds(g * 16, 16)])
            return 0

        lax.fori_loop(0, BE // 16, grp, 0)
        return 0

    lax.fori_loop(0, NBLK, blk, 0)
    pltpu.sync_copy(den_v, dpart_hbm.at[wid])


def _prep_body(x_ref, Wg_ref, as_ref, ad_ref, h_ref, asrc_ref, adst_ref):
    hb = x_ref[...] @ Wg_ref[...]
    h_ref[...] = hb
    asrc_ref[...] = hb @ as_ref[...]
    adst_ref[...] = hb @ ad_ref[...]


def _prep(x, W_gat, att_src, att_dst):
    x_pad = jnp.pad(x, ((0, NPAD - N), (0, 8 - x.shape[1])))
    Wg_pad = jnp.pad(W_gat, ((0, 8 - W_gat.shape[0]), (0, 0)))
    return pl.pallas_call(
        _prep_body,
        grid=(GRID,),
        in_specs=[
            pl.BlockSpec((NB, 8), lambda i: (i, 0)),
            pl.BlockSpec((8, 64), lambda i: (0, 0)),
            pl.BlockSpec((64, 1), lambda i: (0, 0)),
            pl.BlockSpec((64, 1), lambda i: (0, 0)),
        ],
        out_specs=[
            pl.BlockSpec((NB, 64), lambda i: (i, 0)),
            pl.BlockSpec((NB, 1), lambda i: (i, 0)),
            pl.BlockSpec((NB, 1), lambda i: (i, 0)),
        ],
        out_shape=[
            jax.ShapeDtypeStruct((NPAD, 64), jnp.float32),
            jax.ShapeDtypeStruct((NPAD, 1), jnp.float32),
            jax.ShapeDtypeStruct((NPAD, 1), jnp.float32),
        ],
    )(x_pad, Wg_pad, att_src.reshape(64, 1), att_dst.reshape(64, 1))


def _tail_pool_body(acc_ref, h_ref, denom_ref, asrc_ref, adst_ref, mc_ref,
                    bgat_ref, pb_ref, W1_ref, b1_ref, pooled_ref, counts_ref):
    i = pl.program_id(0)

    @pl.when(i == 0)
    def _init():
        pooled_ref[...] = jnp.zeros_like(pooled_ref)
        counts_ref[...] = jnp.zeros_like(counts_ref)

    asel = asrc_ref[...] + adst_ref[...] + mc_ref[...]
    asel = jnp.where(asel >= 0, asel, 0.2 * asel)
    ws = jnp.exp(asel)
    hgat = ((acc_ref[...] + ws * h_ref[...]) / (denom_ref[...] + ws)
            + bgat_ref[...])
    h1 = jnp.maximum(hgat @ W1_ref[...] + b1_ref[...], 0.0)
    pb = pb_ref[0, pl.ds(i * NB, NB)]
    onehot = (pb[None, :] == jax.lax.broadcasted_iota(jnp.int32, (B, NB), 0)
              ).astype(jnp.float32)
    pooled_ref[...] += onehot @ h1
    counts_ref[...] += onehot @ jnp.ones((NB, 1), jnp.float32)


def _head_body(pooled_ref, counts_ref, ag_ref, W2_ref, b2_ref,
               Wv1_ref, bv1_ref, Wv2_ref, bv2_ref,
               Wa1_ref, ba1_ref, Wa2_ref, ba2_ref, out_ref):
    counts = jnp.maximum(counts_ref[...], 1.0)
    pooled = pooled_ref[...] / counts
    ag = jnp.maximum(ag_ref[...] @ W2_ref[...] + b2_ref[...], 0.0)
    z = jnp.concatenate([pooled, ag], axis=-1)
    v = jnp.maximum(z @ Wv1_ref[...] + bv1_ref[...], 0.0)
    value = v @ Wv2_ref[...] + bv2_ref[...]
    a = jnp.maximum(z @ Wa1_ref[...] + ba1_ref[...], 0.0)
    adv = a @ Wa2_ref[...] + ba2_ref[...]
    out_ref[...] = value + adv - jnp.mean(adv)


def _dense_tail(acc_pad, h_pad, denom_pad, asrc2, adst2, mc, b_gat,
                pool_batch, agent_state, W1, b1, W2, b2,
                Wv1, bv1, Wv2, bv2, Wa1, ba1, Wa2, ba2):
    pb_pad = jnp.pad(pool_batch, (0, NPAD - N), constant_values=-1)
    pooled_sum, counts = pl.pallas_call(
        _tail_pool_body,
        grid=(GRID,),
        in_specs=[
            pl.BlockSpec((NB, 64), lambda i: (i, 0)),
            pl.BlockSpec((NB, 64), lambda i: (i, 0)),
            pl.BlockSpec((NB, 1), lambda i: (i, 0)),
            pl.BlockSpec((NB, 1), lambda i: (i, 0)),
            pl.BlockSpec((NB, 1), lambda i: (i, 0)),
            pl.BlockSpec((1, 1), lambda i: (0, 0)),
            pl.BlockSpec((1, 64), lambda i: (0, 0)),
            pl.BlockSpec((1, NPAD), lambda i: (0, 0)),
            pl.BlockSpec((64, 128), lambda i: (0, 0)),
            pl.BlockSpec((1, 128), lambda i: (0, 0)),
        ],
        out_specs=[
            pl.BlockSpec((B, 128), lambda i: (0, 0)),
            pl.BlockSpec((B, 1), lambda i: (0, 0)),
        ],
        out_shape=[
            jax.ShapeDtypeStruct((B, 128), jnp.float32),
            jax.ShapeDtypeStruct((B, 1), jnp.float32),
        ],
    )(acc_pad, h_pad, denom_pad, asrc2, adst2, mc.reshape(1, 1),
      b_gat.reshape(1, 64), pb_pad.reshape(1, NPAD), W1, b1.reshape(1, 128))

    out = pl.pallas_call(
        _head_body,
        out_shape=jax.ShapeDtypeStruct((B, 8), jnp.float32),
    )(pooled_sum, counts, agent_state, W2, b2.reshape(1, 64),
      Wv1, bv1.reshape(1, 128), Wv2, bv2.reshape(1, 1),
      Wa1, ba1.reshape(1, 128), Wa2, ba2.reshape(1, 8))
    return out


def kernel(x, edge_attr, agent_state, edge_index, pool_batch, W_gat, att_src,
           att_dst, W_edge, att_edge, b_gat, W1, b1, W2, b2, Wv1, bv1, Wv2,
           bv2, Wa1, ba1, Wa2, ba2):
    src = edge_index[0]
    dst = edge_index[1]
    c_edge = jnp.dot(W_edge[0], att_edge)
    h_pad, asrc2, adst2 = _prep(x, W_gat, att_src, att_dst)
    a_edge = edge_attr[:, 0] * c_edge

    src_p = jnp.pad(src, (0, EP - E))
    dst_p = jnp.pad(dst, (0, EP - E), constant_values=N)
    ea_p = jnp.pad(a_edge, (0, EP - E))
    asrc_p = asrc2.reshape(NPAD)
    adst_p = adst2.reshape(NPAD)
    t, easum = _k1(src_p, ea_p, asrc_p)
    w_p = _k2(dst_p, t, adst_p)
    dpart = _k2d(dst_p, w_p)
    denom_pad = jnp.sum(dpart, axis=0)
    acc_pad = _k3(src_p, dst_p, w_p, h_pad)[:NPAD]

    mc = jnp.sum(easum).reshape(1) / E

    return _dense_tail(acc_pad, h_pad, denom_pad.reshape(NPAD, 1),
                       asrc2, adst2, mc, b_gat,
                       pool_batch, agent_state, W1, b1, W2, b2,
                       Wv1, bv1, Wv2, bv2, Wa1, ba1, Wa2, ba2)


# speedup vs baseline: 9.2148x; 1.8968x over previous
import functools

import jax
import jax.numpy as jnp
from jax import lax
from jax.experimental import pallas as pl
from jax.experimental.pallas import tpu as pltpu
from jax.experimental.pallas import tpu_sc as plsc

N = 100000
NPAD = 100352
B = 128
NB = 2048
GRID = NPAD // NB

E = 3200000
NW = 32
EP = 3276800
EW = EP // NW
BE = 2048
NBLK = EW // BE

_sc_mesh = plsc.VectorSubcoreMesh(core_axis_name="c", subcore_axis_name="s")
_sc_params = pltpu.CompilerParams(needs_layout_passes=False,
                                  use_tc_tiling_on_sc=False)


def _wid():
    return lax.axis_index("s") * 2 + lax.axis_index("c")


@functools.partial(
    pl.kernel, mesh=_sc_mesh, compiler_params=_sc_params,
    out_type=[jax.ShapeDtypeStruct((EP,), jnp.float32),
              jax.ShapeDtypeStruct((NW, 16), jnp.float32)],
    scratch_types=[
        pltpu.VMEM((NPAD,), jnp.float32),
        pltpu.VMEM((BE,), jnp.int32),
        pltpu.VMEM((BE,), jnp.float32),
        pltpu.VMEM((BE,), jnp.float32),
        pltpu.VMEM((16,), jnp.float32),
    ],
)
def _k1(src_hbm, ea_hbm, asrc_hbm, t_hbm, easum_hbm,
        table_v, src_v, ea_v, t_v, sum_v):
    wid = _wid()
    base = wid * EW
    pltpu.sync_copy(asrc_hbm, table_v)

    def blk(b, acc):
        off = base + b * BE
        pltpu.sync_copy(src_hbm.at[pl.ds(off, BE)], src_v)
        pltpu.sync_copy(ea_hbm.at[pl.ds(off, BE)], ea_v)

        def grp(g, a):
            idx = src_v[pl.ds(g * 16, 16)]
            vals = plsc.load_gather(table_v, [idx])
            ea16 = ea_v[pl.ds(g * 16, 16)]
            t_v[pl.ds(g * 16, 16)] = vals + ea16
            return a + ea16

        acc = lax.fori_loop(0, BE // 16, grp, acc)
        pltpu.sync_copy(t_v, t_hbm.at[pl.ds(off, BE)])
        return acc

    acc = lax.fori_loop(0, NBLK, blk, jnp.zeros((16,), jnp.float32))
    sum_v[...] = acc
    pltpu.sync_copy(sum_v, easum_hbm.at[wid])


@functools.partial(
    pl.kernel, mesh=_sc_mesh, compiler_params=_sc_params,
    out_type=jax.ShapeDtypeStruct((EP,), jnp.float32),
    scratch_types=[
        pltpu.VMEM((NPAD,), jnp.float32),
        pltpu.VMEM((BE,), jnp.int32),
        pltpu.VMEM((BE,), jnp.float32),
        pltpu.VMEM((BE,), jnp.float32),
    ],
)
def _k2(dst_hbm, t_hbm, adst_hbm, w_hbm, table_v, dst_v, t_v, w_v):
    base = _wid() * EW
    pltpu.sync_copy(adst_hbm, table_v)

    def blk(b, _):
        off = base + b * BE
        pltpu.sync_copy(dst_hbm.at[pl.ds(off, BE)], dst_v)
        pltpu.sync_copy(t_hbm.at[pl.ds(off, BE)], t_v)

        def grp(g, _):
            idx = dst_v[pl.ds(g * 16, 16)]
            alpha = plsc.load_gather(table_v, [idx]) + t_v[pl.ds(g * 16, 16)]
            alpha = jnp.where(alpha >= 0, alpha, 0.2 * alpha)
            w_v[pl.ds(g * 16, 16)] = jnp.exp(alpha)
            return 0

        lax.fori_loop(0, BE // 16, grp, 0)
        pltpu.sync_copy(w_v, w_hbm.at[pl.ds(off, BE)])
        return 0

    lax.fori_loop(0, NBLK, blk, 0)


CH = 26624
TROWS = CH // 16
NCHUNK = 4
KQ = 128


@functools.partial(
    pl.kernel, mesh=_sc_mesh, compiler_params=_sc_params,
    out_type=jax.ShapeDtypeStruct((NCHUNK * CH, 64), jnp.float32),
    scratch_types=[
        pltpu.VMEM((BE,), jnp.int32),
        pltpu.VMEM((BE,), jnp.int32),
        pltpu.VMEM((BE,), jnp.float32),
        pltpu.VMEM((KQ, 64), jnp.float32),
        pltpu.VMEM((KQ, 64), jnp.float32),
        pltpu.VMEM((KQ,), jnp.float32),
        pltpu.VMEM((KQ,), jnp.int32),
        pltpu.VMEM_SHARED((CH, 64), jnp.float32),
        pltpu.SemaphoreType.DMA,
    ],
)
def _k3(src_hbm, dst_hbm, w_hbm, h_hbm, acc_hbm,
        src_v, dst_v, w_v, rows_v, zero_v, cw_v, dl_v, acc_sh, sem):
    c = lax.axis_index("c")
    s = lax.axis_index("s")
    base = s * (EP // 16)

    def zrow(i, _):
        for cc in range(4):
            zero_v[i, pl.ds(cc * 16, 16)] = jnp.zeros((16,), jnp.float32)
        return 0

    lax.fori_loop(0, KQ, zrow, 0)

    def scan(lo):
        def blk(b, _):
            off = base + b * BE
            pltpu.sync_copy(src_hbm.at[pl.ds(off, BE)], src_v)
            pltpu.sync_copy(dst_hbm.at[pl.ds(off, BE)], dst_v)
            pltpu.sync_copy(w_hbm.at[pl.ds(off, BE)], w_v)

            def chunk(q, _):
                qo = q * KQ

                def grp(g, _):
                    i = qo + g * 16
                    d16 = dst_v[pl.ds(i, 16)]
                    w16 = w_v[pl.ds(i, 16)]
                    dloc = d16 - lo
                    m = (dloc >= 0) & (dloc < CH)
                    cw_v[pl.ds(g * 16, 16)] = jnp.where(m, w16, 0.0)
                    dl_v[pl.ds(g * 16, 16)] = jnp.where(m, dloc, 0)
                    return 0

                lax.fori_loop(0, KQ // 16, grp, 0)
                pltpu.async_copy(
                    h_hbm.at[src_v.at[pl.ds(qo, KQ)]], rows_v, sem).wait()

                def scale(k, _):
                    wb = plsc.load_gather(
                        cw_v, [jnp.zeros((16,), jnp.int32) + k])
                    for cc in range(4):
                        seg = rows_v[k, pl.ds(cc * 16, 16)]
                        rows_v[k, pl.ds(cc * 16, 16)] = seg * wb
                    return 0

                lax.fori_loop(0, KQ, scale, 0)
                pltpu.sync_copy(rows_v, acc_sh.at[dl_v], add=True)
                return 0

            lax.fori_loop(0, BE // KQ, chunk, 0)
            return 0

        lax.fori_loop(0, (EP // 16) // BE, blk, 0)

    def writeback(lo):
        def wb_loop(z, _):
            row0 = s * TROWS + z * KQ
            pltpu.sync_copy(acc_sh.at[pl.ds(row0, KQ)],
                            acc_hbm.at[pl.ds(lo + row0, KQ)])
            return 0

        lax.fori_loop(0, TROWS // KQ, wb_loop, 0)

    for p in range(2):
        def zacc(z, _):
            pltpu.sync_copy(zero_v, acc_sh.at[pl.ds(s * TROWS + z * KQ, KQ)])
            return 0

        lax.fori_loop(0, TROWS // KQ, zacc, 0)
        plsc.subcore_barrier()

        for cval in range(2):
            @pl.when(c == cval)
            def _(p=p, cval=cval):
                scan((2 * p + cval) * CH)

        plsc.subcore_barrier()

        for cval in range(2):
            @pl.when(c == cval)
            def _(p=p, cval=cval):
                writeback((2 * p + cval) * CH)

        plsc.subcore_barrier()


@functools.partial(
    pl.kernel, mesh=_sc_mesh, compiler_params=_sc_params,
    out_type=jax.ShapeDtypeStruct((NW, NPAD), jnp.float32),
    scratch_types=[
        pltpu.VMEM((NPAD,), jnp.float32),
        pltpu.VMEM((BE,), jnp.int32),
        pltpu.VMEM((BE,), jnp.float32),
    ],
)
def _k2d(dst_hbm, w_hbm, dpart_hbm, den_v, dst_v, w_v):
    wid = _wid()
    base = wid * EW

    def zero(i, _):
        den_v[pl.ds(i * 16, 16)] = jnp.zeros((16,), jnp.float32)
        return 0

    lax.fori_loop(0, NPAD // 16, zero, 0)

    def blk(b, _):
        off = base + b * BE
        pltpu.sync_copy(dst_hbm.at[pl.ds(off, BE)], dst_v)
        pltpu.sync_copy(w_hbm.at[pl.ds(off, BE)], w_v)

        def grp(g, _):
            idx = dst_v[pl.ds(g * 16, 16)]
            plsc.addupdate_scatter(den_v, [idx], w_v[pl.ds(g * 16, 16)])
            return 0

        lax.fori_loop(0, BE // 16, grp, 0)
        return 0

    lax.fori_loop(0, NBLK, blk, 0)
    pltpu.sync_copy(den_v, dpart_hbm.at[wid])


def _prep_body(x_ref, Wg_ref, as_ref, ad_ref, h_ref, asrc_ref, adst_ref):
    hb = x_ref[...] @ Wg_ref[...]
    h_ref[...] = hb
    asrc_ref[...] = hb @ as_ref[...]
    adst_ref[...] = hb @ ad_ref[...]


def _prep(x, W_gat, att_src, att_dst):
    x_pad = jnp.pad(x, ((0, NPAD - N), (0, 8 - x.shape[1])))
    Wg_pad = jnp.pad(W_gat, ((0, 8 - W_gat.shape[0]), (0, 0)))
    return pl.pallas_call(
        _prep_body,
        grid=(GRID,),
        in_specs=[
            pl.BlockSpec((NB, 8), lambda i: (i, 0)),
            pl.BlockSpec((8, 64), lambda i: (0, 0)),
            pl.BlockSpec((64, 1), lambda i: (0, 0)),
            pl.BlockSpec((64, 1), lambda i: (0, 0)),
        ],
        out_specs=[
            pl.BlockSpec((NB, 64), lambda i: (i, 0)),
            pl.BlockSpec((NB, 1), lambda i: (i, 0)),
            pl.BlockSpec((NB, 1), lambda i: (i, 0)),
        ],
        out_shape=[
            jax.ShapeDtypeStruct((NPAD, 64), jnp.float32),
            jax.ShapeDtypeStruct((NPAD, 1), jnp.float32),
            jax.ShapeDtypeStruct((NPAD, 1), jnp.float32),
        ],
    )(x_pad, Wg_pad, att_src.reshape(64, 1), att_dst.reshape(64, 1))


def _tail_pool_body(acc_ref, h_ref, denom_ref, asrc_ref, adst_ref, mc_ref,
                    bgat_ref, pb_ref, W1_ref, b1_ref, pooled_ref, counts_ref):
    i = pl.program_id(0)

    @pl.when(i == 0)
    def _init():
        pooled_ref[...] = jnp.zeros_like(pooled_ref)
        counts_ref[...] = jnp.zeros_like(counts_ref)

    asel = asrc_ref[...] + adst_ref[...] + mc_ref[...]
    asel = jnp.where(asel >= 0, asel, 0.2 * asel)
    ws = jnp.exp(asel)
    hgat = ((acc_ref[...] + ws * h_ref[...]) / (denom_ref[...] + ws)
            + bgat_ref[...])
    h1 = jnp.maximum(hgat @ W1_ref[...] + b1_ref[...], 0.0)
    pb = pb_ref[0, pl.ds(i * NB, NB)]
    onehot = (pb[None, :] == jax.lax.broadcasted_iota(jnp.int32, (B, NB), 0)
              ).astype(jnp.float32)
    pooled_ref[...] += onehot @ h1
    counts_ref[...] += onehot @ jnp.ones((NB, 1), jnp.float32)


def _head_body(pooled_ref, counts_ref, ag_ref, W2_ref, b2_ref,
               Wv1_ref, bv1_ref, Wv2_ref, bv2_ref,
               Wa1_ref, ba1_ref, Wa2_ref, ba2_ref, out_ref):
    counts = jnp.maximum(counts_ref[...], 1.0)
    pooled = pooled_ref[...] / counts
    ag = jnp.maximum(ag_ref[...] @ W2_ref[...] + b2_ref[...], 0.0)
    z = jnp.concatenate([pooled, ag], axis=-1)
    v = jnp.maximum(z @ Wv1_ref[...] + bv1_ref[...], 0.0)
    value = v @ Wv2_ref[...] + bv2_ref[...]
    a = jnp.maximum(z @ Wa1_ref[...] + ba1_ref[...], 0.0)
    adv = a @ Wa2_ref[...] + ba2_ref[...]
    out_ref[...] = value + adv - jnp.mean(adv)


def _dense_tail(acc_pad, h_pad, denom_pad, asrc2, adst2, mc, b_gat,
                pool_batch, agent_state, W1, b1, W2, b2,
                Wv1, bv1, Wv2, bv2, Wa1, ba1, Wa2, ba2):
    pb_pad = jnp.pad(pool_batch, (0, NPAD - N), constant_values=-1)
    pooled_sum, counts = pl.pallas_call(
        _tail_pool_body,
        grid=(GRID,),
        in_specs=[
            pl.BlockSpec((NB, 64), lambda i: (i, 0)),
            pl.BlockSpec((NB, 64), lambda i: (i, 0)),
            pl.BlockSpec((NB, 1), lambda i: (i, 0)),
            pl.BlockSpec((NB, 1), lambda i: (i, 0)),
            pl.BlockSpec((NB, 1), lambda i: (i, 0)),
            pl.BlockSpec((1, 1), lambda i: (0, 0)),
            pl.BlockSpec((1, 64), lambda i: (0, 0)),
            pl.BlockSpec((1, NPAD), lambda i: (0, 0)),
            pl.BlockSpec((64, 128), lambda i: (0, 0)),
            pl.BlockSpec((1, 128), lambda i: (0, 0)),
        ],
        out_specs=[
            pl.BlockSpec((B, 128), lambda i: (0, 0)),
            pl.BlockSpec((B, 1), lambda i: (0, 0)),
        ],
        out_shape=[
            jax.ShapeDtypeStruct((B, 128), jnp.float32),
            jax.ShapeDtypeStruct((B, 1), jnp.float32),
        ],
    )(acc_pad, h_pad, denom_pad, asrc2, adst2, mc.reshape(1, 1),
      b_gat.reshape(1, 64), pb_pad.reshape(1, NPAD), W1, b1.reshape(1, 128))

    out = pl.pallas_call(
        _head_body,
        out_shape=jax.ShapeDtypeStruct((B, 8), jnp.float32),
    )(pooled_sum, counts, agent_state, W2, b2.reshape(1, 64),
      Wv1, bv1.reshape(1, 128), Wv2, bv2.reshape(1, 1),
      Wa1, ba1.reshape(1, 128), Wa2, ba2.reshape(1, 8))
    return out


def kernel(x, edge_attr, agent_state, edge_index, pool_batch, W_gat, att_src,
           att_dst, W_edge, att_edge, b_gat, W1, b1, W2, b2, Wv1, bv1, Wv2,
           bv2, Wa1, ba1, Wa2, ba2):
    src = edge_index[0]
    dst = edge_index[1]
    c_edge = jnp.dot(W_edge[0], att_edge)
    h_pad, asrc2, adst2 = _prep(x, W_gat, att_src, att_dst)
    a_edge = edge_attr[:, 0] * c_edge

    src_p = jnp.pad(src, (0, EP - E))
    dst_p = jnp.pad(dst, (0, EP - E), constant_values=N)
    ea_p = jnp.pad(a_edge, (0, EP - E))
    asrc_p = asrc2.reshape(NPAD)
    adst_p = adst2.reshape(NPAD)
    t, easum = _k1(src_p, ea_p, asrc_p)
    w_p = _k2(dst_p, t, adst_p)
    dpart = _k2d(dst_p, w_p)
    denom_pad = jnp.sum(dpart, axis=0)
    acc_pad = _k3(src_p, dst_p, w_p, h_pad)[:NPAD]

    mc = jnp.sum(easum).reshape(1) / E

    return _dense_tail(acc_pad, h_pad, denom_pad.reshape(NPAD, 1),
                       asrc2, adst2, mc, b_gat,
                       pool_batch, agent_state, W1, b1, W2, b2,
                       Wv1, bv1, Wv2, bv2, Wa1, ba1, Wa2, ba2)


# speedup vs baseline: 10.2675x; 1.1142x over previous
import functools

import jax
import jax.numpy as jnp
from jax import lax
from jax.experimental import pallas as pl
from jax.experimental.pallas import tpu as pltpu
from jax.experimental.pallas import tpu_sc as plsc

N = 100000
NPAD = 100352
B = 128
NB = 2048
GRID = NPAD // NB

E = 3200000
NW = 32
EP = 3276800
EW = EP // NW
BE = 2048
NBLK = EW // BE

_sc_mesh = plsc.VectorSubcoreMesh(core_axis_name="c", subcore_axis_name="s")
_sc_params = pltpu.CompilerParams(needs_layout_passes=False,
                                  use_tc_tiling_on_sc=False)


def _wid():
    return lax.axis_index("s") * 2 + lax.axis_index("c")


@functools.partial(
    pl.kernel, mesh=_sc_mesh, compiler_params=_sc_params,
    out_type=[jax.ShapeDtypeStruct((EP,), jnp.float32),
              jax.ShapeDtypeStruct((NW, 16), jnp.float32)],
    scratch_types=[
        pltpu.VMEM((NPAD,), jnp.float32),
        pltpu.VMEM((BE,), jnp.int32),
        pltpu.VMEM((BE,), jnp.float32),
        pltpu.VMEM((BE,), jnp.float32),
        pltpu.VMEM((16,), jnp.float32),
    ],
)
def _k1(src_hbm, ea_hbm, asrc_hbm, t_hbm, easum_hbm,
        table_v, src_v, ea_v, t_v, sum_v):
    wid = _wid()
    base = wid * EW
    pltpu.sync_copy(asrc_hbm, table_v)

    def blk(b, acc):
        off = base + b * BE
        pltpu.sync_copy(src_hbm.at[pl.ds(off, BE)], src_v)
        pltpu.sync_copy(ea_hbm.at[pl.ds(off, BE)], ea_v)

        def grp(g, a):
            idx = src_v[pl.ds(g * 16, 16)]
            vals = plsc.load_gather(table_v, [idx])
            ea16 = ea_v[pl.ds(g * 16, 16)]
            t_v[pl.ds(g * 16, 16)] = vals + ea16
            return a + ea16

        acc = lax.fori_loop(0, BE // 16, grp, acc)
        pltpu.sync_copy(t_v, t_hbm.at[pl.ds(off, BE)])
        return acc

    acc = lax.fori_loop(0, NBLK, blk, jnp.zeros((16,), jnp.float32))
    sum_v[...] = acc
    pltpu.sync_copy(sum_v, easum_hbm.at[wid])


@functools.partial(
    pl.kernel, mesh=_sc_mesh, compiler_params=_sc_params,
    out_type=jax.ShapeDtypeStruct((EP,), jnp.float32),
    scratch_types=[
        pltpu.VMEM((NPAD,), jnp.float32),
        pltpu.VMEM((BE,), jnp.int32),
        pltpu.VMEM((BE,), jnp.float32),
        pltpu.VMEM((BE,), jnp.float32),
    ],
)
def _k2(dst_hbm, t_hbm, adst_hbm, w_hbm, table_v, dst_v, t_v, w_v):
    base = _wid() * EW
    pltpu.sync_copy(adst_hbm, table_v)

    def blk(b, _):
        off = base + b * BE
        pltpu.sync_copy(dst_hbm.at[pl.ds(off, BE)], dst_v)
        pltpu.sync_copy(t_hbm.at[pl.ds(off, BE)], t_v)

        def grp(g, _):
            idx = dst_v[pl.ds(g * 16, 16)]
            alpha = plsc.load_gather(table_v, [idx]) + t_v[pl.ds(g * 16, 16)]
            alpha = jnp.where(alpha >= 0, alpha, 0.2 * alpha)
            w_v[pl.ds(g * 16, 16)] = jnp.exp(alpha)
            return 0

        lax.fori_loop(0, BE // 16, grp, 0)
        pltpu.sync_copy(w_v, w_hbm.at[pl.ds(off, BE)])
        return 0

    lax.fori_loop(0, NBLK, blk, 0)


CH = 25088
TROWS = CH // 16
NCHUNK = 4
KQ = 64


@functools.partial(
    pl.kernel, mesh=_sc_mesh, compiler_params=_sc_params,
    out_type=jax.ShapeDtypeStruct((NCHUNK * CH, 64), jnp.float32),
    scratch_types=[
        pltpu.VMEM((BE,), jnp.int32),
        pltpu.VMEM((BE,), jnp.int32),
        pltpu.VMEM((BE,), jnp.float32),
        pltpu.VMEM((KQ, 64), jnp.float32),
        pltpu.VMEM((KQ, 64), jnp.float32),
        pltpu.VMEM((KQ, 64), jnp.float32),
        pltpu.VMEM((BE,), jnp.float32),
        pltpu.VMEM((BE // KQ, KQ), jnp.int32),
        pltpu.VMEM_SHARED((CH + 8, 64), jnp.float32),
        pltpu.SemaphoreType.DMA,
        pltpu.SemaphoreType.DMA,
        pltpu.SemaphoreType.DMA,
        pltpu.SemaphoreType.DMA,
    ],
)
def _k3(src_hbm, dst_hbm, w_hbm, h_hbm, acc_hbm,
        src_v, dst_v, w_v, rows0_v, rows1_v, zero_v, cw_v, dl_v, acc_sh,
        gsem0, gsem1, ssem0, ssem1):
    c = lax.axis_index("c")
    s = lax.axis_index("s")
    base = s * (EP // 16)

    def zrow(i, _):
        for cc in range(4):
            zero_v[i, pl.ds(cc * 16, 16)] = jnp.zeros((16,), jnp.float32)
        return 0

    lax.fori_loop(0, KQ, zrow, 0)

    def scan(lo):
        NCHK = BE // KQ
        rows = (rows0_v, rows1_v)
        gsem = (gsem0, gsem1)
        ssem = (ssem0, ssem1)

        def blk(b, _):
            off = base + b * BE
            pltpu.sync_copy(src_hbm.at[pl.ds(off, BE)], src_v)
            pltpu.sync_copy(dst_hbm.at[pl.ds(off, BE)], dst_v)
            pltpu.sync_copy(w_hbm.at[pl.ds(off, BE)], w_v)

            def grp(g, _):
                d16 = dst_v[pl.ds(g * 16, 16)]
                w16 = w_v[pl.ds(g * 16, 16)]
                dloc = d16 - lo
                m = (dloc >= 0) & (dloc < CH)
                cw_v[pl.ds(g * 16, 16)] = jnp.where(m, w16, 0.0)
                gpc = KQ // 16
                dl_v[g // gpc, pl.ds((g % gpc) * 16, 16)] = jnp.where(m, dloc, 0)
                return 0

            lax.fori_loop(0, BE // 16, grp, 0)

            pltpu.async_copy(h_hbm.at[src_v.at[pl.ds(0, KQ)]],
                             rows[0], gsem[0])
            for q in range(NCHK):
                cur = q % 2
                oth = 1 - cur
                if q + 1 < NCHK:
                    if q >= 1:
                        pltpu.make_async_copy(
                            rows[oth], acc_sh.at[dl_v.at[q - 1]],
                            ssem[oth]).wait()
                    pltpu.async_copy(
                        h_hbm.at[src_v.at[pl.ds((q + 1) * KQ, KQ)]],
                        rows[oth], gsem[oth])
                pltpu.make_async_copy(
                    h_hbm.at[src_v.at[pl.ds(q * KQ, KQ)]],
                    rows[cur], gsem[cur]).wait()

                def scale(k, _):
                    wb = plsc.load_gather(
                        cw_v, [jnp.zeros((16,), jnp.int32) + (q * KQ + k)])
                    for cc in range(4):
                        seg = rows[cur][k, pl.ds(cc * 16, 16)]
                        rows[cur][k, pl.ds(cc * 16, 16)] = seg * wb
                    return 0

                lax.fori_loop(0, KQ, scale, 0)
                pltpu.async_copy(rows[cur], acc_sh.at[dl_v.at[q]],
                                 ssem[cur], add=True)

            pltpu.make_async_copy(
                rows[0], acc_sh.at[dl_v.at[NCHK - 2]], ssem[0]).wait()
            pltpu.make_async_copy(
                rows[1], acc_sh.at[dl_v.at[NCHK - 1]], ssem[1]).wait()
            return 0

        lax.fori_loop(0, (EP // 16) // BE, blk, 0)

    ZB = 32

    def writeback(lo):
        def wb_loop(z, _):
            row0 = s * TROWS + z * ZB
            pltpu.sync_copy(acc_sh.at[pl.ds(row0, ZB)],
                            acc_hbm.at[pl.ds(lo + row0, ZB)])
            return 0

        lax.fori_loop(0, TROWS // ZB, wb_loop, 0)

    for p in range(2):
        def zacc(z, _):
            pltpu.sync_copy(zero_v.at[pl.ds(0, ZB)],
                            acc_sh.at[pl.ds(s * TROWS + z * ZB, ZB)])
            return 0

        lax.fori_loop(0, TROWS // ZB, zacc, 0)
        plsc.subcore_barrier()

        for cval in range(2):
            @pl.when(c == cval)
            def _(p=p, cval=cval):
                scan((2 * p + cval) * CH)

        plsc.subcore_barrier()

        for cval in range(2):
            @pl.when(c == cval)
            def _(p=p, cval=cval):
                writeback((2 * p + cval) * CH)

        plsc.subcore_barrier()


@functools.partial(
    pl.kernel, mesh=_sc_mesh, compiler_params=_sc_params,
    out_type=jax.ShapeDtypeStruct((NW, NPAD), jnp.float32),
    scratch_types=[
        pltpu.VMEM((NPAD,), jnp.float32),
        pltpu.VMEM((BE,), jnp.int32),
        pltpu.VMEM((BE,), jnp.float32),
    ],
)
def _k2d(dst_hbm, w_hbm, dpart_hbm, den_v, dst_v, w_v):
    wid = _wid()
    base = wid * EW

    def zero(i, _):
        den_v[pl.ds(i * 16, 16)] = jnp.zeros((16,), jnp.float32)
        return 0

    lax.fori_loop(0, NPAD // 16, zero, 0)

    def blk(b, _):
        off = base + b * BE
        pltpu.sync_copy(dst_hbm.at[pl.ds(off, BE)], dst_v)
        pltpu.sync_copy(w_hbm.at[pl.ds(off, BE)], w_v)

        def grp(g, _):
            idx = dst_v[pl.ds(g * 16, 16)]
            plsc.addupdate_scatter(den_v, [idx], w_v[pl.ds(g * 16, 16)])
            return 0

        lax.fori_loop(0, BE // 16, grp, 0)
        return 0

    lax.fori_loop(0, NBLK, blk, 0)
    pltpu.sync_copy(den_v, dpart_hbm.at[wid])


def _prep_body(x_ref, Wg_ref, as_ref, ad_ref, h_ref, asrc_ref, adst_ref):
    hb = x_ref[...] @ Wg_ref[...]
    h_ref[...] = hb
    asrc_ref[...] = hb @ as_ref[...]
    adst_ref[...] = hb @ ad_ref[...]


def _prep(x, W_gat, att_src, att_dst):
    x_pad = jnp.pad(x, ((0, NPAD - N), (0, 8 - x.shape[1])))
    Wg_pad = jnp.pad(W_gat, ((0, 8 - W_gat.shape[0]), (0, 0)))
    return pl.pallas_call(
        _prep_body,
        grid=(GRID,),
        in_specs=[
            pl.BlockSpec((NB, 8), lambda i: (i, 0)),
            pl.BlockSpec((8, 64), lambda i: (0, 0)),
            pl.BlockSpec((64, 1), lambda i: (0, 0)),
            pl.BlockSpec((64, 1), lambda i: (0, 0)),
        ],
        out_specs=[
            pl.BlockSpec((NB, 64), lambda i: (i, 0)),
            pl.BlockSpec((NB, 1), lambda i: (i, 0)),
            pl.BlockSpec((NB, 1), lambda i: (i, 0)),
        ],
        out_shape=[
            jax.ShapeDtypeStruct((NPAD, 64), jnp.float32),
            jax.ShapeDtypeStruct((NPAD, 1), jnp.float32),
            jax.ShapeDtypeStruct((NPAD, 1), jnp.float32),
        ],
    )(x_pad, Wg_pad, att_src.reshape(64, 1), att_dst.reshape(64, 1))


def _tail_pool_body(acc_ref, h_ref, denom_ref, asrc_ref, adst_ref, mc_ref,
                    bgat_ref, pb_ref, W1_ref, b1_ref, pooled_ref, counts_ref):
    i = pl.program_id(0)

    @pl.when(i == 0)
    def _init():
        pooled_ref[...] = jnp.zeros_like(pooled_ref)
        counts_ref[...] = jnp.zeros_like(counts_ref)

    asel = asrc_ref[...] + adst_ref[...] + mc_ref[...]
    asel = jnp.where(asel >= 0, asel, 0.2 * asel)
    ws = jnp.exp(asel)
    hgat = ((acc_ref[...] + ws * h_ref[...]) / (denom_ref[...] + ws)
            + bgat_ref[...])
    h1 = jnp.maximum(hgat @ W1_ref[...] + b1_ref[...], 0.0)
    pb = pb_ref[0, pl.ds(i * NB, NB)]
    onehot = (pb[None, :] == jax.lax.broadcasted_iota(jnp.int32, (B, NB), 0)
              ).astype(jnp.float32)
    pooled_ref[...] += onehot @ h1
    counts_ref[...] += onehot @ jnp.ones((NB, 1), jnp.float32)


def _head_body(pooled_ref, counts_ref, ag_ref, W2_ref, b2_ref,
               Wv1_ref, bv1_ref, Wv2_ref, bv2_ref,
               Wa1_ref, ba1_ref, Wa2_ref, ba2_ref, out_ref):
    counts = jnp.maximum(counts_ref[...], 1.0)
    pooled = pooled_ref[...] / counts
    ag = jnp.maximum(ag_ref[...] @ W2_ref[...] + b2_ref[...], 0.0)
    z = jnp.concatenate([pooled, ag], axis=-1)
    v = jnp.maximum(z @ Wv1_ref[...] + bv1_ref[...], 0.0)
    value = v @ Wv2_ref[...] + bv2_ref[...]
    a = jnp.maximum(z @ Wa1_ref[...] + ba1_ref[...], 0.0)
    adv = a @ Wa2_ref[...] + ba2_ref[...]
    out_ref[...] = value + adv - jnp.mean(adv)


def _dense_tail(acc_pad, h_pad, denom_pad, asrc2, adst2, mc, b_gat,
                pool_batch, agent_state, W1, b1, W2, b2,
                Wv1, bv1, Wv2, bv2, Wa1, ba1, Wa2, ba2):
    pb_pad = jnp.pad(pool_batch, (0, NPAD - N), constant_values=-1)
    pooled_sum, counts = pl.pallas_call(
        _tail_pool_body,
        grid=(GRID,),
        in_specs=[
            pl.BlockSpec((NB, 64), lambda i: (i, 0)),
            pl.BlockSpec((NB, 64), lambda i: (i, 0)),
            pl.BlockSpec((NB, 1), lambda i: (i, 0)),
            pl.BlockSpec((NB, 1), lambda i: (i, 0)),
            pl.BlockSpec((NB, 1), lambda i: (i, 0)),
            pl.BlockSpec((1, 1), lambda i: (0, 0)),
            pl.BlockSpec((1, 64), lambda i: (0, 0)),
            pl.BlockSpec((1, NPAD), lambda i: (0, 0)),
            pl.BlockSpec((64, 128), lambda i: (0, 0)),
            pl.BlockSpec((1, 128), lambda i: (0, 0)),
        ],
        out_specs=[
            pl.BlockSpec((B, 128), lambda i: (0, 0)),
            pl.BlockSpec((B, 1), lambda i: (0, 0)),
        ],
        out_shape=[
            jax.ShapeDtypeStruct((B, 128), jnp.float32),
            jax.ShapeDtypeStruct((B, 1), jnp.float32),
        ],
    )(acc_pad, h_pad, denom_pad, asrc2, adst2, mc.reshape(1, 1),
      b_gat.reshape(1, 64), pb_pad.reshape(1, NPAD), W1, b1.reshape(1, 128))

    out = pl.pallas_call(
        _head_body,
        out_shape=jax.ShapeDtypeStruct((B, 8), jnp.float32),
    )(pooled_sum, counts, agent_state, W2, b2.reshape(1, 64),
      Wv1, bv1.reshape(1, 128), Wv2, bv2.reshape(1, 1),
      Wa1, ba1.reshape(1, 128), Wa2, ba2.reshape(1, 8))
    return out


def kernel(x, edge_attr, agent_state, edge_index, pool_batch, W_gat, att_src,
           att_dst, W_edge, att_edge, b_gat, W1, b1, W2, b2, Wv1, bv1, Wv2,
           bv2, Wa1, ba1, Wa2, ba2):
    src = edge_index[0]
    dst = edge_index[1]
    c_edge = jnp.dot(W_edge[0], att_edge)
    h_pad, asrc2, adst2 = _prep(x, W_gat, att_src, att_dst)
    a_edge = edge_attr[:, 0] * c_edge

    src_p = jnp.pad(src, (0, EP - E))
    dst_p = jnp.pad(dst, (0, EP - E), constant_values=N)
    ea_p = jnp.pad(a_edge, (0, EP - E))
    asrc_p = asrc2.reshape(NPAD)
    adst_p = adst2.reshape(NPAD)
    t, easum = _k1(src_p, ea_p, asrc_p)
    w_p = _k2(dst_p, t, adst_p)
    dpart = _k2d(dst_p, w_p)
    denom_pad = jnp.sum(dpart, axis=0)
    acc_pad = _k3(src_p, dst_p, w_p, h_pad)[:NPAD]

    mc = jnp.sum(easum).reshape(1) / E

    return _dense_tail(acc_pad, h_pad, denom_pad.reshape(NPAD, 1),
                       asrc2, adst2, mc, b_gat,
                       pool_batch, agent_state, W1, b1, W2, b2,
                       Wv1, bv1, Wv2, bv2, Wa1, ba1, Wa2, ba2)
